# baseline (speedup 1.0000x reference)
import functools

import jax
import jax.numpy as jnp
from jax import lax
from jax.experimental import pallas as pl
from jax.experimental.pallas import tpu as pltpu
from jax.experimental.pallas import tpu_sc as plsc

S_OUT = 1024
K_NN = 32
BIG_F32 = 3.0e38


def _fps_body(xyz_ref, out_ref, *, n_samples):
    x = xyz_ref[:, 0, :]
    y = xyz_ref[:, 1, :]
    z = xyz_ref[:, 2, :]
    Bv, Nv = x.shape
    iota = jax.lax.broadcasted_iota(jnp.int32, (Bv, Nv), 1)

    def body(i, carry):
        dists, far = carry
        mask = (iota == far).astype(jnp.float32)
        cx = jnp.sum(x * mask, axis=1, keepdims=True)
        cy = jnp.sum(y * mask, axis=1, keepdims=True)
        cz = jnp.sum(z * mask, axis=1, keepdims=True)
        dx = x - cx
        dy = y - cy
        dz = z - cz
        d = (dx * dx + dy * dy) + dz * dz
        dists = jnp.minimum(dists, d)
        m = jnp.max(dists, axis=1, keepdims=True)
        nxt = jnp.min(jnp.where(dists == m, iota, Nv), axis=1, keepdims=True)
        out_ref[:, pl.ds(i, 1), :] = jnp.concatenate(
            [cx[:, :, None], cy[:, :, None], cz[:, :, None]], axis=2)
        return (dists, nxt.astype(jnp.int32))

    dists0 = jnp.full((Bv, Nv), 1e10, dtype=jnp.float32)
    far0 = jnp.zeros((Bv, 1), dtype=jnp.int32)
    jax.lax.fori_loop(0, n_samples, body, (dists0, far0))


def _fps(xyz, n_samples):
    B, _, N = xyz.shape
    return pl.pallas_call(
        functools.partial(_fps_body, n_samples=n_samples),
        out_shape=jax.ShapeDtypeStruct((B, n_samples, 3), jnp.float32),
    )(xyz)


def _knn_body(samp_ref, xyz_ref, out_ref, *, k_nn):
    s = samp_ref[0]
    sx = s[:, 0:1]
    sy = s[:, 1:2]
    sz = s[:, 2:3]
    x = xyz_ref[0, 0, :][None, :]
    y = xyz_ref[0, 1, :][None, :]
    z = xyz_ref[0, 2, :][None, :]
    Nv = x.shape[1]
    Sb = s.shape[0]
    iota = jax.lax.broadcasted_iota(jnp.int32, (Sb, Nv), 1)

    sq_s = (sx * sx + sy * sy) + sz * sz
    sq_p = (x * x + y * y) + z * z
    cross = jnp.dot(s.astype(jnp.bfloat16), xyz_ref[0].astype(jnp.bfloat16),
                    preferred_element_type=jnp.float32)
    d2 = (sq_s - 2.0 * cross) + sq_p

    sels = []
    for _ in range(k_nn):
        m = jnp.min(d2, axis=1, keepdims=True)
        sel = jnp.min(jnp.where(d2 == m, iota, Nv), axis=1, keepdims=True)
        sels.append(sel)
        d2 = jnp.where(iota == sel, BIG_F32, d2)
    out_ref[0] = jnp.concatenate(sels, axis=1)


def _knn_idx(sampled, xyz, k_nn, sb):
    B, S, _ = sampled.shape
    N = xyz.shape[2]
    return pl.pallas_call(
        functools.partial(_knn_body, k_nn=k_nn),
        grid=(B, S // sb),
        in_specs=[
            pl.BlockSpec((1, sb, 3), lambda b, s: (b, s, 0)),
            pl.BlockSpec((1, 3, N), lambda b, s: (b, 0, 0)),
        ],
        out_specs=pl.BlockSpec((1, sb, k_nn), lambda b, s: (b, s, 0)),
        out_shape=jax.ShapeDtypeStruct((B, S, k_nn), jnp.int32),
    )(sampled, xyz)


_GCHUNK = 128
_PAD = 16


def _sc_gather(idx, xyz):
    B, _, N = xyz.shape
    P = idx.shape[0]
    info = plsc.get_sparse_core_info()
    nw = info.num_cores * info.num_subcores
    per_w = P // nw
    wpb = nw // B
    nchunk = per_w // _GCHUNK
    mesh = plsc.VectorSubcoreMesh(core_axis_name="c", subcore_axis_name="s")

    @functools.partial(
        pl.kernel,
        out_type=jax.ShapeDtypeStruct((P * _PAD,), jnp.float32),
        mesh=mesh,
        compiler_params=pltpu.CompilerParams(needs_layout_passes=False),
        scratch_types=[
            pltpu.VMEM((N,), jnp.float32),
            pltpu.VMEM((N,), jnp.float32),
            pltpu.VMEM((N,), jnp.float32),
            pltpu.VMEM((_GCHUNK,), jnp.int32),
            pltpu.VMEM((_GCHUNK * _PAD,), jnp.float32),
        ],
    )
    def gath(idx_hbm, xyz_hbm, out_hbm, xv, yv, zv, idx_v, buf):
        wid = lax.axis_index("s") * info.num_cores + lax.axis_index("c")
        b = wid // wpb
        base = wid * per_w
        pltpu.sync_copy(xyz_hbm.at[pl.ds((b * 3 + 0) * N, N)], xv)
        pltpu.sync_copy(xyz_hbm.at[pl.ds((b * 3 + 1) * N, N)], yv)
        pltpu.sync_copy(xyz_hbm.at[pl.ds((b * 3 + 2) * N, N)], zv)
        lane = lax.iota(jnp.int32, 16)

        @pl.loop(0, nchunk)
        def _(j):
            off = base + j * _GCHUNK
            pltpu.sync_copy(idx_hbm.at[pl.ds(off, _GCHUNK)], idx_v)
            for i in range(_GCHUNK // 16):
                iv = idx_v[pl.ds(i * 16, 16)]
                gx = plsc.load_gather(xv, [iv])
                gy = plsc.load_gather(yv, [iv])
                gz = plsc.load_gather(zv, [iv])
                pos = (lane + i * 16) * _PAD
                plsc.store_scatter(buf, [pos], gx)
                plsc.store_scatter(buf, [pos + 1], gy)
                plsc.store_scatter(buf, [pos + 2], gz)
            pltpu.sync_copy(buf, out_hbm.at[pl.ds(off * _PAD, _GCHUNK * _PAD)])

    return gath(idx, xyz.reshape(B * 3 * N))


def _center_mm_stats_body(xg_ref, q_ref, w_ref, b_ref, y_ref, st_ref, *, k_nn):
    xg = xg_ref[...][:, 0:3]
    q = q_ref[...]
    qrep = jnp.broadcast_to(q[:, None, :], (q.shape[0], k_nn, 3)).reshape(-1, 3)
    x = xg - qrep
    y = jnp.dot(x, w_ref[...], preferred_element_type=jnp.float32) + b_ref[...][None, :]
    y_ref[...] = y
    st_ref[0, 0, :] = jnp.sum(y, axis=0)
    st_ref[0, 1, :] = jnp.sum(y * y, axis=0)


def _center_mm_stats(xg, q, w_t, b, k_nn, pb):
    P, D = xg.shape
    Cout = w_t.shape[1]
    nblk = P // pb
    return pl.pallas_call(
        functools.partial(_center_mm_stats_body, k_nn=k_nn),
        grid=(nblk,),
        in_specs=[
            pl.BlockSpec((pb, D), lambda i: (i, 0)),
            pl.BlockSpec((pb // k_nn, 3), lambda i: (i, 0)),
            pl.BlockSpec((3, Cout), lambda i: (0, 0)),
            pl.BlockSpec((Cout,), lambda i: (0,)),
        ],
        out_specs=[
            pl.BlockSpec((pb, Cout), lambda i: (i, 0)),
            pl.BlockSpec((1, 2, Cout), lambda i: (i, 0, 0)),
        ],
        out_shape=[
            jax.ShapeDtypeStruct((P, Cout), jnp.float32),
            jax.ShapeDtypeStruct((nblk, 2, Cout), jnp.float32),
        ],
    )(xg, q, w_t, b)


def _bnrelu_mm_stats_body(x_ref, a_ref, c_ref, w_ref, b_ref, y_ref, st_ref):
    h = jax.nn.relu(x_ref[...] * a_ref[...][None, :] + c_ref[...][None, :])
    y = jnp.dot(h, w_ref[...], preferred_element_type=jnp.float32) + b_ref[...][None, :]
    y_ref[...] = y
    st_ref[0, 0, :] = jnp.sum(y, axis=0)
    st_ref[0, 1, :] = jnp.sum(y * y, axis=0)


def _bnrelu_mm_stats(x, a, c, w_t, b, pb):
    P, Cin = x.shape
    Cout = w_t.shape[1]
    nblk = P // pb
    return pl.pallas_call(
        _bnrelu_mm_stats_body,
        grid=(nblk,),
        in_specs=[
            pl.BlockSpec((pb, Cin), lambda i: (i, 0)),
            pl.BlockSpec((Cin,), lambda i: (0,)),
            pl.BlockSpec((Cin,), lambda i: (0,)),
            pl.BlockSpec((Cin, Cout), lambda i: (0, 0)),
            pl.BlockSpec((Cout,), lambda i: (0,)),
        ],
        out_specs=[
            pl.BlockSpec((pb, Cout), lambda i: (i, 0)),
            pl.BlockSpec((1, 2, Cout), lambda i: (i, 0, 0)),
        ],
        out_shape=[
            jax.ShapeDtypeStruct((P, Cout), jnp.float32),
            jax.ShapeDtypeStruct((nblk, 2, Cout), jnp.float32),
        ],
    )(x, a, c, w_t, b)


def _bnrelu_mm_max_body(x_ref, a_ref, c_ref, w_ref, b_ref, o_ref, *, k_nn):
    h = jax.nn.relu(x_ref[...] * a_ref[...][None, :] + c_ref[...][None, :])
    y = jnp.dot(h, w_ref[...], preferred_element_type=jnp.float32) + b_ref[...][None, :]
    pb, Cout = y.shape
    y3 = y.reshape(pb // k_nn, k_nn, Cout)
    acc = y3[:, 0, :]
    for k in range(1, k_nn):
        acc = jnp.maximum(acc, y3[:, k, :])
    o_ref[...] = acc


def _bnrelu_mm_max(x, a, c, w_t, b, k_nn, pb):
    P, Cin = x.shape
    Cout = w_t.shape[1]
    nblk = P // pb
    return pl.pallas_call(
        functools.partial(_bnrelu_mm_max_body, k_nn=k_nn),
        grid=(nblk,),
        in_specs=[
            pl.BlockSpec((pb, Cin), lambda i: (i, 0)),
            pl.BlockSpec((Cin,), lambda i: (0,)),
            pl.BlockSpec((Cin,), lambda i: (0,)),
            pl.BlockSpec((Cin, Cout), lambda i: (0, 0)),
            pl.BlockSpec((Cout,), lambda i: (0,)),
        ],
        out_specs=pl.BlockSpec((pb // k_nn, Cout), lambda i: (i, 0)),
        out_shape=jax.ShapeDtypeStruct((P // k_nn, Cout), jnp.float32),
    )(x, a, c, w_t, b)


def _bn_coeffs(partials, count, g, be):
    s = jnp.sum(partials[:, 0, :], axis=0)
    ss = jnp.sum(partials[:, 1, :], axis=0)
    mu = s / count
    var = ss / count - mu * mu
    a = g / jnp.sqrt(var + 1e-5)
    c = be - mu * a
    return a, c


def kernel(xyz, W1, b1, g1, be1, W2, b2, g2, be2, Wl, bl):
    B, _, N = xyz.shape
    sampled = _fps(xyz, S_OUT)
    knn_idx = _knn_idx(sampled, xyz, K_NN, 64)
    P = B * S_OUT * K_NN
    gathered = _sc_gather(knn_idx.reshape(P), xyz).reshape(P, _PAD)
    y1, st1 = _center_mm_stats(gathered, sampled.reshape(B * S_OUT, 3),
                               W1.T, b1, K_NN, 4096)
    a1, c1 = _bn_coeffs(st1, P, g1, be1)
    y2, st2 = _bnrelu_mm_stats(y1, a1, c1, W2.T, b2, 4096)
    a2, c2 = _bn_coeffs(st2, P, g2, be2)
    out = _bnrelu_mm_max(y2, a2, c2, Wl.T, bl, K_NN, 4096)
    new_points = out.reshape(B, S_OUT, -1).transpose(0, 2, 1)
    sampled_xyz = sampled.transpose(0, 2, 1)
    return (sampled_xyz, new_points)


# speedup vs baseline: 1.2110x; 1.2110x over previous
import functools

import jax
import jax.numpy as jnp
from jax import lax
from jax.experimental import pallas as pl
from jax.experimental.pallas import tpu as pltpu
from jax.experimental.pallas import tpu_sc as plsc

S_OUT = 1024
K_NN = 32
BIG_F32 = 3.0e38


_FPS_UNROLL = 8


def _fps_body(xyz_ref, out_ref, *, n_samples):
    x = xyz_ref[:, 0]
    y = xyz_ref[:, 1]
    z = xyz_ref[:, 2]
    Bv, R, C = x.shape
    Nv = R * C
    iota = (jax.lax.broadcasted_iota(jnp.int32, (Bv, R, C), 1) * C
            + jax.lax.broadcasted_iota(jnp.int32, (Bv, R, C), 2))

    def _rmin(v):
        return jnp.min(jnp.min(v, axis=2, keepdims=True), axis=1, keepdims=True)

    def _rmax(v):
        return jnp.max(jnp.max(v, axis=2, keepdims=True), axis=1, keepdims=True)

    def _rsum(v):
        return jnp.sum(jnp.sum(v, axis=2, keepdims=True), axis=1, keepdims=True)

    def body(i, carry):
        dists, far = carry
        cols = []
        for _ in range(_FPS_UNROLL):
            mask = (iota == far).astype(jnp.float32)
            cx = _rsum(x * mask)
            cy = _rsum(y * mask)
            cz = _rsum(z * mask)
            dx = x - cx
            dy = y - cy
            dz = z - cz
            d = (dx * dx + dy * dy) + dz * dz
            dists = jnp.minimum(dists, d)
            m = _rmax(dists)
            far = _rmin(jnp.where(dists == m, iota, Nv)).astype(jnp.int32)
            cols.append(jnp.concatenate([cx[:, 0], cy[:, 0], cz[:, 0]], axis=1))
        block = jnp.stack(cols, axis=1)
        out_ref[:, pl.ds(i * _FPS_UNROLL, _FPS_UNROLL), :] = block
        return (dists, far)

    dists0 = jnp.full((Bv, R, C), 1e10, dtype=jnp.float32)
    far0 = jnp.zeros((Bv, 1, 1), dtype=jnp.int32)
    jax.lax.fori_loop(0, n_samples // _FPS_UNROLL, body, (dists0, far0))


def _fps(xyz, n_samples):
    B, _, N = xyz.shape
    return pl.pallas_call(
        functools.partial(_fps_body, n_samples=n_samples),
        out_shape=jax.ShapeDtypeStruct((B, n_samples, 3), jnp.float32),
    )(xyz.reshape(B, 3, 8, N // 8))


def _knn_body(samp_ref, xyz_ref, out_ref, *, k_nn):
    s = samp_ref[0]
    sx = s[:, 0:1]
    sy = s[:, 1:2]
    sz = s[:, 2:3]
    x = xyz_ref[0, 0, :][None, :]
    y = xyz_ref[0, 1, :][None, :]
    z = xyz_ref[0, 2, :][None, :]
    Nv = x.shape[1]
    Sb = s.shape[0]
    iota = jax.lax.broadcasted_iota(jnp.int32, (Sb, Nv), 1)

    sq_s = (sx * sx + sy * sy) + sz * sz
    sq_p = (x * x + y * y) + z * z
    cross = jnp.dot(s.astype(jnp.bfloat16), xyz_ref[0].astype(jnp.bfloat16),
                    preferred_element_type=jnp.float32)
    d2 = (sq_s - 2.0 * cross) + sq_p

    sels = []
    for _ in range(k_nn):
        m = jnp.min(d2, axis=1, keepdims=True)
        sel = jnp.min(jnp.where(d2 == m, iota, Nv), axis=1, keepdims=True)
        sels.append(sel)
        d2 = jnp.where(iota == sel, BIG_F32, d2)
    out_ref[0] = jnp.concatenate(sels, axis=1)


def _knn_idx(sampled, xyz, k_nn, sb):
    B, S, _ = sampled.shape
    N = xyz.shape[2]
    return pl.pallas_call(
        functools.partial(_knn_body, k_nn=k_nn),
        grid=(B, S // sb),
        in_specs=[
            pl.BlockSpec((1, sb, 3), lambda b, s: (b, s, 0)),
            pl.BlockSpec((1, 3, N), lambda b, s: (b, 0, 0)),
        ],
        out_specs=pl.BlockSpec((1, sb, k_nn), lambda b, s: (b, s, 0)),
        out_shape=jax.ShapeDtypeStruct((B, S, k_nn), jnp.int32),
    )(sampled, xyz)


_GCHUNK = 128
_PAD = 16


def _sc_gather(idx, xyz):
    B, _, N = xyz.shape
    P = idx.shape[0]
    info = plsc.get_sparse_core_info()
    nw = info.num_cores * info.num_subcores
    per_w = P // nw
    wpb = nw // B
    nchunk = per_w // _GCHUNK
    mesh = plsc.VectorSubcoreMesh(core_axis_name="c", subcore_axis_name="s")

    @functools.partial(
        pl.kernel,
        out_type=jax.ShapeDtypeStruct((P * _PAD,), jnp.float32),
        mesh=mesh,
        compiler_params=pltpu.CompilerParams(needs_layout_passes=False),
        scratch_types=[
            pltpu.VMEM((N,), jnp.float32),
            pltpu.VMEM((N,), jnp.float32),
            pltpu.VMEM((N,), jnp.float32),
            pltpu.VMEM((_GCHUNK,), jnp.int32),
            pltpu.VMEM((_GCHUNK * _PAD,), jnp.float32),
        ],
    )
    def gath(idx_hbm, xyz_hbm, out_hbm, xv, yv, zv, idx_v, buf):
        wid = lax.axis_index("s") * info.num_cores + lax.axis_index("c")
        b = wid // wpb
        base = wid * per_w
        pltpu.sync_copy(xyz_hbm.at[pl.ds((b * 3 + 0) * N, N)], xv)
        pltpu.sync_copy(xyz_hbm.at[pl.ds((b * 3 + 1) * N, N)], yv)
        pltpu.sync_copy(xyz_hbm.at[pl.ds((b * 3 + 2) * N, N)], zv)
        lane = lax.iota(jnp.int32, 16)

        @pl.loop(0, nchunk)
        def _(j):
            off = base + j * _GCHUNK
            pltpu.sync_copy(idx_hbm.at[pl.ds(off, _GCHUNK)], idx_v)
            for i in range(_GCHUNK // 16):
                iv = idx_v[pl.ds(i * 16, 16)]
                gx = plsc.load_gather(xv, [iv])
                gy = plsc.load_gather(yv, [iv])
                gz = plsc.load_gather(zv, [iv])
                pos = (lane + i * 16) * _PAD
                plsc.store_scatter(buf, [pos], gx)
                plsc.store_scatter(buf, [pos + 1], gy)
                plsc.store_scatter(buf, [pos + 2], gz)
            pltpu.sync_copy(buf, out_hbm.at[pl.ds(off * _PAD, _GCHUNK * _PAD)])

    return gath(idx, xyz.reshape(B * 3 * N))


def _center_mm_stats_body(xg_ref, q_ref, w_ref, b_ref, y_ref, st_ref, *, k_nn):
    xg = xg_ref[...][:, 0:3]
    q = q_ref[...]
    qrep = jnp.broadcast_to(q[:, None, :], (q.shape[0], k_nn, 3)).reshape(-1, 3)
    x = xg - qrep
    y = jnp.dot(x, w_ref[...], preferred_element_type=jnp.float32) + b_ref[...][None, :]
    y_ref[...] = y
    st_ref[0, 0, :] = jnp.sum(y, axis=0)
    st_ref[0, 1, :] = jnp.sum(y * y, axis=0)


def _center_mm_stats(xg, q, w_t, b, k_nn, pb):
    P, D = xg.shape
    Cout = w_t.shape[1]
    nblk = P // pb
    return pl.pallas_call(
        functools.partial(_center_mm_stats_body, k_nn=k_nn),
        grid=(nblk,),
        in_specs=[
            pl.BlockSpec((pb, D), lambda i: (i, 0)),
            pl.BlockSpec((pb // k_nn, 3), lambda i: (i, 0)),
            pl.BlockSpec((3, Cout), lambda i: (0, 0)),
            pl.BlockSpec((Cout,), lambda i: (0,)),
        ],
        out_specs=[
            pl.BlockSpec((pb, Cout), lambda i: (i, 0)),
            pl.BlockSpec((1, 2, Cout), lambda i: (i, 0, 0)),
        ],
        out_shape=[
            jax.ShapeDtypeStruct((P, Cout), jnp.float32),
            jax.ShapeDtypeStruct((nblk, 2, Cout), jnp.float32),
        ],
    )(xg, q, w_t, b)


def _bnrelu_mm_stats_body(x_ref, a_ref, c_ref, w_ref, b_ref, y_ref, st_ref):
    h = jax.nn.relu(x_ref[...] * a_ref[...][None, :] + c_ref[...][None, :])
    y = jnp.dot(h, w_ref[...], preferred_element_type=jnp.float32) + b_ref[...][None, :]
    y_ref[...] = y
    st_ref[0, 0, :] = jnp.sum(y, axis=0)
    st_ref[0, 1, :] = jnp.sum(y * y, axis=0)


def _bnrelu_mm_stats(x, a, c, w_t, b, pb):
    P, Cin = x.shape
    Cout = w_t.shape[1]
    nblk = P // pb
    return pl.pallas_call(
        _bnrelu_mm_stats_body,
        grid=(nblk,),
        in_specs=[
            pl.BlockSpec((pb, Cin), lambda i: (i, 0)),
            pl.BlockSpec((Cin,), lambda i: (0,)),
            pl.BlockSpec((Cin,), lambda i: (0,)),
            pl.BlockSpec((Cin, Cout), lambda i: (0, 0)),
            pl.BlockSpec((Cout,), lambda i: (0,)),
        ],
        out_specs=[
            pl.BlockSpec((pb, Cout), lambda i: (i, 0)),
            pl.BlockSpec((1, 2, Cout), lambda i: (i, 0, 0)),
        ],
        out_shape=[
            jax.ShapeDtypeStruct((P, Cout), jnp.float32),
            jax.ShapeDtypeStruct((nblk, 2, Cout), jnp.float32),
        ],
    )(x, a, c, w_t, b)


def _bnrelu_mm_max_body(x_ref, a_ref, c_ref, w_ref, b_ref, o_ref, *, k_nn):
    h = jax.nn.relu(x_ref[...] * a_ref[...][None, :] + c_ref[...][None, :])
    y = jnp.dot(h, w_ref[...], preferred_element_type=jnp.float32) + b_ref[...][None, :]
    pb, Cout = y.shape
    y3 = y.reshape(pb // k_nn, k_nn, Cout)
    acc = y3[:, 0, :]
    for k in range(1, k_nn):
        acc = jnp.maximum(acc, y3[:, k, :])
    o_ref[...] = acc


def _bnrelu_mm_max(x, a, c, w_t, b, k_nn, pb):
    P, Cin = x.shape
    Cout = w_t.shape[1]
    nblk = P // pb
    return pl.pallas_call(
        functools.partial(_bnrelu_mm_max_body, k_nn=k_nn),
        grid=(nblk,),
        in_specs=[
            pl.BlockSpec((pb, Cin), lambda i: (i, 0)),
            pl.BlockSpec((Cin,), lambda i: (0,)),
            pl.BlockSpec((Cin,), lambda i: (0,)),
            pl.BlockSpec((Cin, Cout), lambda i: (0, 0)),
            pl.BlockSpec((Cout,), lambda i: (0,)),
        ],
        out_specs=pl.BlockSpec((pb // k_nn, Cout), lambda i: (i, 0)),
        out_shape=jax.ShapeDtypeStruct((P // k_nn, Cout), jnp.float32),
    )(x, a, c, w_t, b)


def _bn_coeffs(partials, count, g, be):
    s = jnp.sum(partials[:, 0, :], axis=0)
    ss = jnp.sum(partials[:, 1, :], axis=0)
    mu = s / count
    var = ss / count - mu * mu
    a = g / jnp.sqrt(var + 1e-5)
    c = be - mu * a
    return a, c


def kernel(xyz, W1, b1, g1, be1, W2, b2, g2, be2, Wl, bl):
    B, _, N = xyz.shape
    sampled = _fps(xyz, S_OUT)
    knn_idx = _knn_idx(sampled, xyz, K_NN, 32)
    P = B * S_OUT * K_NN
    gathered = _sc_gather(knn_idx.reshape(P), xyz).reshape(P, _PAD)
    y1, st1 = _center_mm_stats(gathered, sampled.reshape(B * S_OUT, 3),
                               W1.T, b1, K_NN, 4096)
    a1, c1 = _bn_coeffs(st1, P, g1, be1)
    y2, st2 = _bnrelu_mm_stats(y1, a1, c1, W2.T, b2, 4096)
    a2, c2 = _bn_coeffs(st2, P, g2, be2)
    out = _bnrelu_mm_max(y2, a2, c2, Wl.T, bl, K_NN, 4096)
    new_points = out.reshape(B, S_OUT, -1).transpose(0, 2, 1)
    sampled_xyz = sampled.transpose(0, 2, 1)
    return (sampled_xyz, new_points)
